# baseline (speedup 1.0000x reference)
import functools

import jax
import jax.numpy as jnp
from jax.experimental import pallas as pl
from jax.experimental.pallas import tpu as pltpu


def _router_kernel(x_ref, wr_ref, top1_ref, top2_ref, g1_ref, g2_ref):
    x = x_ref[...]
    w = wr_ref[...]
    logits = jnp.dot(x, w.T, preferred_element_type=jnp.float32)
    E = logits.shape[1]
    TM2 = logits.shape[0]

    iota_e = jax.lax.broadcasted_iota(jnp.int32, (TM2, E), 1)
    neg = jnp.float32(-jnp.inf)
    best_v = jnp.max(logits, axis=1, keepdims=True)
    best_i = jnp.min(jnp.where(logits == best_v, iota_e, E), axis=1,
                     keepdims=True)
    masked = jnp.where(iota_e == best_i, neg, logits)
    sec_v = jnp.max(masked, axis=1, keepdims=True)
    sec_i = jnp.min(jnp.where(masked == sec_v, iota_e, E), axis=1,
                    keepdims=True)

    z = jnp.exp(sec_v - best_v)
    g1 = 1.0 / (1.0 + z)
    g2 = 1.0 - g1
    top1_ref[...] = best_i
    top2_ref[...] = sec_i
    g1_ref[...] = g1
    g2_ref[...] = g2


def _run_router(xf, W_router, TM2):
    T, D = xf.shape
    E = W_router.shape[0]
    grid = (T // TM2,)
    out_shape = [
        jax.ShapeDtypeStruct((T, 1), jnp.int32),
        jax.ShapeDtypeStruct((T, 1), jnp.int32),
        jax.ShapeDtypeStruct((T, 1), jnp.float32),
        jax.ShapeDtypeStruct((T, 1), jnp.float32),
    ]
    o_spec = pl.BlockSpec((TM2, 1), lambda i: (i, 0))
    return pl.pallas_call(
        _router_kernel,
        grid=grid,
        in_specs=[
            pl.BlockSpec((TM2, D), lambda i: (i, 0)),
            pl.BlockSpec((E, D), lambda i: (0, 0)),
        ],
        out_specs=[o_spec, o_spec, o_spec, o_spec],
        out_shape=out_shape,
    )(xf, W_router)


def _gmm_kernel(st_ref, se_ref, isf_ref, isl_ref, sstart_ref, send_ref,
                x_ref, wg_ref, wi_ref, wo_ref, gates_ref, out_ref, acc_ref):
    g = pl.program_id(0)
    j = pl.program_id(1)
    J = pl.num_programs(1)
    TM = x_ref.shape[0]

    x = x_ref[...]
    h1 = jnp.dot(x, wg_ref[0], preferred_element_type=jnp.float32)
    h2 = jnp.dot(x, wi_ref[0], preferred_element_type=jnp.float32)

    rows = st_ref[g] * TM + jax.lax.broadcasted_iota(jnp.int32, (TM, 1), 0)
    valid = (rows >= sstart_ref[g]) & (rows < send_ref[g])
    gate = jnp.where(valid, gates_ref[...], 0.0)

    act = (jax.nn.silu(h1) * h2 * gate).astype(jnp.bfloat16)
    y = jnp.dot(act, wo_ref[0], preferred_element_type=jnp.float32)

    first = (isf_ref[g] == 1) & (j == 0)

    @pl.when(first)
    def _init():
        acc_ref[...] = y

    @pl.when(jnp.logical_not(first))
    def _acc():
        acc_ref[...] += y

    @pl.when((isl_ref[g] == 1) & (j == J - 1))
    def _emit():
        out_ref[...] = acc_ref[...].astype(jnp.bfloat16)


def _run_gmm(xs, gates_sorted, W_in, W_out, st, se, isf, isl, sstart, send,
             TM, NJ):
    P, D = xs.shape
    E, _, MOD_IN = W_in.shape
    MOD_OUT = W_out.shape[1]
    J = MOD_OUT // NJ
    G = st.shape[0]

    grid_spec = pltpu.PrefetchScalarGridSpec(
        num_scalar_prefetch=6,
        grid=(G, J),
        in_specs=[
            pl.BlockSpec((TM, D), lambda g, j, st, *_: (st[g], 0)),
            pl.BlockSpec((1, D, NJ),
                         lambda g, j, st, se, *_: (se[g], 0, j)),
            pl.BlockSpec((1, D, NJ),
                         lambda g, j, st, se, *_, _J=J: (se[g], 0, _J + j)),
            pl.BlockSpec((1, NJ, D),
                         lambda g, j, st, se, *_: (se[g], j, 0)),
            pl.BlockSpec((TM, 1), lambda g, j, st, *_: (st[g], 0)),
        ],
        out_specs=pl.BlockSpec((TM, D), lambda g, j, st, *_: (st[g], 0)),
        scratch_shapes=[pltpu.VMEM((TM, D), jnp.float32)],
    )
    return pl.pallas_call(
        _gmm_kernel,
        grid_spec=grid_spec,
        out_shape=jax.ShapeDtypeStruct((P, D), jnp.bfloat16),
    )(st, se, isf, isl, sstart, send, xs, W_in, W_in, W_out, gates_sorted)


def kernel(x, W_router, W_in, W_out):
    Bx, Sx, D = x.shape
    T = Bx * Sx
    E = W_router.shape[0]
    MOD_OUT = W_out.shape[1]
    P = 2 * T

    TM = min(256, P)
    NJ = min(1024, MOD_OUT)
    TM2 = min(1024, T)
    num_tiles = P // TM

    xf = x.reshape(T, D)
    top1, top2, g1, g2 = _run_router(xf, W_router.astype(jnp.float32), TM2)
    top1 = top1[:, 0]
    top2 = top2[:, 0]

    pair_expert = jnp.concatenate([top1, top2])
    pair_gate = jnp.concatenate([g1[:, 0], g2[:, 0]])
    sort_idx = jnp.argsort(pair_expert).astype(jnp.int32)
    token_sorted = jnp.where(sort_idx >= T, sort_idx - T, sort_idx)
    gates_sorted = pair_gate[sort_idx][:, None]
    pos = jnp.argsort(sort_idx).astype(jnp.int32)

    sizes = jnp.bincount(pair_expert, length=E).astype(jnp.int32)
    ends = jnp.cumsum(sizes).astype(jnp.int32)
    starts = ends - sizes
    tile_lo = starts // TM
    tile_hi = jnp.where(sizes > 0, (ends - 1) // TM, tile_lo - 1)
    spg = jnp.maximum(tile_hi - tile_lo + 1, 0)
    spg_end = jnp.cumsum(spg)
    spg_start = spg_end - spg

    G = num_tiles + E - 1
    gidx = jnp.arange(G, dtype=jnp.int32)
    se = jnp.searchsorted(spg_end, gidx, side='right').astype(jnp.int32)
    se_c = jnp.minimum(se, E - 1)
    valid_step = gidx < spg_end[-1]
    st = jnp.where(valid_step, tile_lo[se_c] + (gidx - spg_start[se_c]),
                   num_tiles - 1).astype(jnp.int32)
    isf = jnp.concatenate([jnp.ones((1,), jnp.int32),
                           (st[1:] != st[:-1]).astype(jnp.int32)])
    isl = jnp.concatenate([(st[1:] != st[:-1]).astype(jnp.int32),
                           jnp.ones((1,), jnp.int32)])
    sstart = jnp.where(valid_step, starts[se_c], 0).astype(jnp.int32)
    send = jnp.where(valid_step, ends[se_c], 0).astype(jnp.int32)

    xs = jnp.take(xf.astype(jnp.bfloat16), token_sorted, axis=0)
    out_sorted = _run_gmm(xs, gates_sorted,
                          W_in.astype(jnp.bfloat16), W_out.astype(jnp.bfloat16),
                          st, se_c, isf, isl, sstart, send, TM, NJ)

    out_pairs = jnp.take(out_sorted, pos, axis=0)
    out = out_pairs[:T].astype(jnp.float32) + out_pairs[T:].astype(jnp.float32)
    return out.reshape(Bx, Sx, D)


# speedup vs baseline: 1.0906x; 1.0906x over previous
import functools

import jax
import jax.numpy as jnp
from jax.experimental import pallas as pl
from jax.experimental.pallas import tpu as pltpu


def _router_kernel(x_ref, wr_ref, top1_ref, top2_ref, g1_ref, g2_ref):
    x = x_ref[...]
    w = wr_ref[...]
    logits = jnp.dot(x, w.T, preferred_element_type=jnp.float32)
    E = logits.shape[1]
    TM2 = logits.shape[0]

    iota_e = jax.lax.broadcasted_iota(jnp.int32, (TM2, E), 1)
    neg = jnp.float32(-jnp.inf)
    best_v = jnp.max(logits, axis=1, keepdims=True)
    best_i = jnp.min(jnp.where(logits == best_v, iota_e, E), axis=1,
                     keepdims=True)
    masked = jnp.where(iota_e == best_i, neg, logits)
    sec_v = jnp.max(masked, axis=1, keepdims=True)
    sec_i = jnp.min(jnp.where(masked == sec_v, iota_e, E), axis=1,
                    keepdims=True)

    z = jnp.exp(sec_v - best_v)
    g1 = 1.0 / (1.0 + z)
    g2 = 1.0 - g1
    top1_ref[...] = best_i
    top2_ref[...] = sec_i
    g1_ref[...] = g1
    g2_ref[...] = g2


def _run_router(xf, W_router, TM2):
    T, D = xf.shape
    E = W_router.shape[0]
    grid = (T // TM2,)
    out_shape = [
        jax.ShapeDtypeStruct((T, 1), jnp.int32),
        jax.ShapeDtypeStruct((T, 1), jnp.int32),
        jax.ShapeDtypeStruct((T, 1), jnp.float32),
        jax.ShapeDtypeStruct((T, 1), jnp.float32),
    ]
    o_spec = pl.BlockSpec((TM2, 1), lambda i: (i, 0))
    return pl.pallas_call(
        _router_kernel,
        grid=grid,
        in_specs=[
            pl.BlockSpec((TM2, D), lambda i: (i, 0)),
            pl.BlockSpec((E, D), lambda i: (0, 0)),
        ],
        out_specs=[o_spec, o_spec, o_spec, o_spec],
        out_shape=out_shape,
    )(xf, W_router)


def _gmm_kernel(st_ref, se_ref, isf_ref, isl_ref, sstart_ref, send_ref,
                x_ref, wg_ref, wi_ref, wo_ref, gates_ref, out_ref, acc_ref):
    g = pl.program_id(0)
    j = pl.program_id(1)
    J = pl.num_programs(1)
    TM = x_ref.shape[0]

    x = x_ref[...]
    h1 = jnp.dot(x, wg_ref[0], preferred_element_type=jnp.float32)
    h2 = jnp.dot(x, wi_ref[0], preferred_element_type=jnp.float32)

    rows = st_ref[g] * TM + jax.lax.broadcasted_iota(jnp.int32, (TM, 1), 0)
    valid = (rows >= sstart_ref[g]) & (rows < send_ref[g])
    gate = jnp.where(valid, gates_ref[...], 0.0)

    act = (jax.nn.silu(h1) * h2 * gate).astype(jnp.bfloat16)
    y = jnp.dot(act, wo_ref[0], preferred_element_type=jnp.float32)

    first = (isf_ref[g] == 1) & (j == 0)

    @pl.when(first)
    def _init():
        acc_ref[...] = y

    @pl.when(jnp.logical_not(first))
    def _acc():
        acc_ref[...] += y

    @pl.when((isl_ref[g] == 1) & (j == J - 1))
    def _emit():
        out_ref[...] = acc_ref[...].astype(jnp.bfloat16)


def _run_gmm(xs, gates_sorted, W_in, W_out, st, se, isf, isl, sstart, send,
             TM, NJ):
    P, D = xs.shape
    E, _, MOD_IN = W_in.shape
    MOD_OUT = W_out.shape[1]
    J = MOD_OUT // NJ
    G = st.shape[0]

    grid_spec = pltpu.PrefetchScalarGridSpec(
        num_scalar_prefetch=6,
        grid=(G, J),
        in_specs=[
            pl.BlockSpec((TM, D), lambda g, j, st, *_: (st[g], 0)),
            pl.BlockSpec((1, D, NJ),
                         lambda g, j, st, se, *_: (se[g], 0, j)),
            pl.BlockSpec((1, D, NJ),
                         lambda g, j, st, se, *_, _J=J: (se[g], 0, _J + j)),
            pl.BlockSpec((1, NJ, D),
                         lambda g, j, st, se, *_: (se[g], j, 0)),
            pl.BlockSpec((TM, 1), lambda g, j, st, *_: (st[g], 0)),
        ],
        out_specs=pl.BlockSpec((TM, D), lambda g, j, st, *_: (st[g], 0)),
        scratch_shapes=[pltpu.VMEM((TM, D), jnp.float32)],
    )
    return pl.pallas_call(
        _gmm_kernel,
        grid_spec=grid_spec,
        out_shape=jax.ShapeDtypeStruct((P, D), jnp.bfloat16),
    )(st, se, isf, isl, sstart, send, xs, W_in, W_in, W_out, gates_sorted)


def kernel(x, W_router, W_in, W_out):
    Bx, Sx, D = x.shape
    T = Bx * Sx
    E = W_router.shape[0]
    MOD_OUT = W_out.shape[1]
    P = 2 * T

    TM = min(512, P)
    NJ = min(1024, MOD_OUT)
    TM2 = min(1024, T)
    num_tiles = P // TM

    xf = x.reshape(T, D)
    top1, top2, g1, g2 = _run_router(xf, W_router.astype(jnp.float32), TM2)
    top1 = top1[:, 0]
    top2 = top2[:, 0]

    pair_expert = jnp.concatenate([top1, top2])
    pair_gate = jnp.concatenate([g1[:, 0], g2[:, 0]])
    sort_idx = jnp.argsort(pair_expert).astype(jnp.int32)
    token_sorted = jnp.where(sort_idx >= T, sort_idx - T, sort_idx)
    gates_sorted = pair_gate[sort_idx][:, None]
    pos = jnp.argsort(sort_idx).astype(jnp.int32)

    sizes = jnp.bincount(pair_expert, length=E).astype(jnp.int32)
    ends = jnp.cumsum(sizes).astype(jnp.int32)
    starts = ends - sizes
    tile_lo = starts // TM
    tile_hi = jnp.where(sizes > 0, (ends - 1) // TM, tile_lo - 1)
    spg = jnp.maximum(tile_hi - tile_lo + 1, 0)
    spg_end = jnp.cumsum(spg)
    spg_start = spg_end - spg

    G = num_tiles + E - 1
    gidx = jnp.arange(G, dtype=jnp.int32)
    se = jnp.searchsorted(spg_end, gidx, side='right').astype(jnp.int32)
    se_c = jnp.minimum(se, E - 1)
    valid_step = gidx < spg_end[-1]
    st = jnp.where(valid_step, tile_lo[se_c] + (gidx - spg_start[se_c]),
                   num_tiles - 1).astype(jnp.int32)
    isf = jnp.concatenate([jnp.ones((1,), jnp.int32),
                           (st[1:] != st[:-1]).astype(jnp.int32)])
    isl = jnp.concatenate([(st[1:] != st[:-1]).astype(jnp.int32),
                           jnp.ones((1,), jnp.int32)])
    sstart = jnp.where(valid_step, starts[se_c], 0).astype(jnp.int32)
    send = jnp.where(valid_step, ends[se_c], 0).astype(jnp.int32)

    xs = jnp.take(xf.astype(jnp.bfloat16), token_sorted, axis=0)
    out_sorted = _run_gmm(xs, gates_sorted,
                          W_in.astype(jnp.bfloat16), W_out.astype(jnp.bfloat16),
                          st, se_c, isf, isl, sstart, send, TM, NJ)

    out_pairs = jnp.take(out_sorted, pos, axis=0)
    out = out_pairs[:T].astype(jnp.float32) + out_pairs[T:].astype(jnp.float32)
    return out.reshape(Bx, Sx, D)


# speedup vs baseline: 1.1173x; 1.0245x over previous
import functools

import jax
import jax.numpy as jnp
from jax.experimental import pallas as pl
from jax.experimental.pallas import tpu as pltpu


def _router_kernel(x_ref, wr_ref, top1_ref, top2_ref, g1_ref, g2_ref):
    x = x_ref[...]
    w = wr_ref[...]
    logits = jnp.dot(x, w.T, preferred_element_type=jnp.float32)
    E = logits.shape[1]
    TM2 = logits.shape[0]

    iota_e = jax.lax.broadcasted_iota(jnp.int32, (TM2, E), 1)
    neg = jnp.float32(-jnp.inf)
    best_v = jnp.max(logits, axis=1, keepdims=True)
    best_i = jnp.min(jnp.where(logits == best_v, iota_e, E), axis=1,
                     keepdims=True)
    masked = jnp.where(iota_e == best_i, neg, logits)
    sec_v = jnp.max(masked, axis=1, keepdims=True)
    sec_i = jnp.min(jnp.where(masked == sec_v, iota_e, E), axis=1,
                    keepdims=True)

    z = jnp.exp(sec_v - best_v)
    g1 = 1.0 / (1.0 + z)
    g2 = 1.0 - g1
    top1_ref[...] = best_i
    top2_ref[...] = sec_i
    g1_ref[...] = g1
    g2_ref[...] = g2


def _run_router(xf, W_router, TM2):
    T, D = xf.shape
    E = W_router.shape[0]
    grid = (T // TM2,)
    out_shape = [
        jax.ShapeDtypeStruct((T, 1), jnp.int32),
        jax.ShapeDtypeStruct((T, 1), jnp.int32),
        jax.ShapeDtypeStruct((T, 1), jnp.float32),
        jax.ShapeDtypeStruct((T, 1), jnp.float32),
    ]
    o_spec = pl.BlockSpec((TM2, 1), lambda i: (i, 0))
    return pl.pallas_call(
        _router_kernel,
        grid=grid,
        in_specs=[
            pl.BlockSpec((TM2, D), lambda i: (i, 0)),
            pl.BlockSpec((E, D), lambda i: (0, 0)),
        ],
        out_specs=[o_spec, o_spec, o_spec, o_spec],
        out_shape=out_shape,
    )(xf, W_router)


def _gmm_kernel(st_ref, se_ref, isf_ref, isl_ref, sstart_ref, send_ref,
                x_ref, wg_ref, wi_ref, wo_ref, gates_ref, out_ref, acc_ref):
    g = pl.program_id(0)
    j = pl.program_id(1)
    J = pl.num_programs(1)
    TM = x_ref.shape[0]

    x = x_ref[...]
    h1 = jnp.dot(x, wg_ref[0], preferred_element_type=jnp.float32)
    h2 = jnp.dot(x, wi_ref[0], preferred_element_type=jnp.float32)

    rows = st_ref[g] * TM + jax.lax.broadcasted_iota(jnp.int32, (TM, 1), 0)
    valid = (rows >= sstart_ref[g]) & (rows < send_ref[g])
    gate = jnp.where(valid, gates_ref[...], 0.0)

    act = (jax.nn.silu(h1) * h2 * gate).astype(jnp.bfloat16)
    y = jnp.dot(act, wo_ref[0], preferred_element_type=jnp.float32)

    first = (isf_ref[g] == 1) & (j == 0)

    @pl.when(first)
    def _init():
        acc_ref[...] = y

    @pl.when(jnp.logical_not(first))
    def _acc():
        acc_ref[...] += y

    @pl.when((isl_ref[g] == 1) & (j == J - 1))
    def _emit():
        out_ref[...] = acc_ref[...].astype(jnp.bfloat16)


def _run_gmm(xs, gates_sorted, W_in, W_out, st, se, isf, isl, sstart, send,
             TM, NJ):
    P, D = xs.shape
    E, _, MOD_IN = W_in.shape
    MOD_OUT = W_out.shape[1]
    J = MOD_OUT // NJ
    G = st.shape[0]

    grid_spec = pltpu.PrefetchScalarGridSpec(
        num_scalar_prefetch=6,
        grid=(G, J),
        in_specs=[
            pl.BlockSpec((TM, D), lambda g, j, st, *_: (st[g], 0)),
            pl.BlockSpec((1, D, NJ),
                         lambda g, j, st, se, *_: (se[g], 0, j)),
            pl.BlockSpec((1, D, NJ),
                         lambda g, j, st, se, *_, _J=J: (se[g], 0, _J + j)),
            pl.BlockSpec((1, NJ, D),
                         lambda g, j, st, se, *_: (se[g], j, 0)),
            pl.BlockSpec((TM, 1), lambda g, j, st, *_: (st[g], 0)),
        ],
        out_specs=pl.BlockSpec((TM, D), lambda g, j, st, *_: (st[g], 0)),
        scratch_shapes=[pltpu.VMEM((TM, D), jnp.float32)],
    )
    return pl.pallas_call(
        _gmm_kernel,
        grid_spec=grid_spec,
        out_shape=jax.ShapeDtypeStruct((P, D), jnp.bfloat16),
    )(st, se, isf, isl, sstart, send, xs, W_in, W_in, W_out, gates_sorted)


def kernel(x, W_router, W_in, W_out):
    Bx, Sx, D = x.shape
    T = Bx * Sx
    E = W_router.shape[0]
    MOD_OUT = W_out.shape[1]
    P = 2 * T

    TM = min(512, P)
    NJ = min(1024, MOD_OUT)
    TM2 = min(1024, T)
    num_tiles = P // TM

    xf = x.reshape(T, D)
    top1, top2, g1, g2 = _run_router(xf, W_router.astype(jnp.float32), TM2)
    top1 = top1[:, 0]
    top2 = top2[:, 0]

    pair_expert = jnp.concatenate([top1, top2])
    pair_gate = jnp.concatenate([g1[:, 0], g2[:, 0]])
    sort_idx = jnp.argsort(pair_expert).astype(jnp.int32)
    token_sorted = jnp.where(sort_idx >= T, sort_idx - T, sort_idx)
    gates_sorted = pair_gate[sort_idx][:, None]
    pos = jnp.argsort(sort_idx).astype(jnp.int32)

    sizes = jnp.bincount(pair_expert, length=E).astype(jnp.int32)
    ends = jnp.cumsum(sizes).astype(jnp.int32)
    starts = ends - sizes
    tile_lo = starts // TM
    tile_hi = jnp.where(sizes > 0, (ends - 1) // TM, tile_lo - 1)
    spg = jnp.maximum(tile_hi - tile_lo + 1, 0)
    spg_end = jnp.cumsum(spg)
    spg_start = spg_end - spg

    G = num_tiles + E - 1
    gidx = jnp.arange(G, dtype=jnp.int32)
    se = jnp.searchsorted(spg_end, gidx, side='right').astype(jnp.int32)
    se_c = jnp.minimum(se, E - 1)
    valid_step = gidx < spg_end[-1]
    st = jnp.where(valid_step, tile_lo[se_c] + (gidx - spg_start[se_c]),
                   num_tiles - 1).astype(jnp.int32)
    isf = jnp.concatenate([jnp.ones((1,), jnp.int32),
                           (st[1:] != st[:-1]).astype(jnp.int32)])
    isl = jnp.concatenate([(st[1:] != st[:-1]).astype(jnp.int32),
                           jnp.ones((1,), jnp.int32)])
    sstart = jnp.where(valid_step, starts[se_c], 0).astype(jnp.int32)
    send = jnp.where(valid_step, ends[se_c], 0).astype(jnp.int32)

    xs = jnp.take(xf.astype(jnp.bfloat16), token_sorted, axis=0)
    out_sorted = _run_gmm(xs, gates_sorted,
                          W_in.astype(jnp.bfloat16), W_out.astype(jnp.bfloat16),
                          st, se_c, isf, isl, sstart, send, TM, NJ)

    out_a = jnp.take(out_sorted, pos[:T], axis=0)
    out_b = jnp.take(out_sorted, pos[T:], axis=0)
    out = out_a.astype(jnp.float32) + out_b.astype(jnp.float32)
    return out.reshape(Bx, Sx, D)
